# baseline (speedup 1.0000x reference)
import functools

import jax
import jax.numpy as jnp
from jax import lax
from jax.experimental import pallas as pl
from jax.experimental.pallas import tpu as pltpu
from jax.experimental.pallas import tpu_sc as plsc


def _table_body(x_ref, nt_ref, wcat_ref, tcat_ref, out_ref):
    acc = jnp.dot(x_ref[...], wcat_ref[...],
                  preferred_element_type=jnp.float32,
                  precision=lax.Precision.HIGHEST)
    nt = nt_ref[...]
    for t in range(4):
        mask = jnp.where(nt == t, 1.0, 0.0)
        acc = acc + mask * tcat_ref[t:t + 1, :]
    out_ref[...] = acc


def _make_sc_edge_kernel(n_tab_words, n_edges):
    nc, ns = 2, 16
    nw = nc * ns
    assert n_edges % 128 == 0
    n_blk = n_edges // 128
    blk_lo = n_blk // nw
    n_hi = n_blk - blk_lo * nw
    cb = 16
    ec = cb * 128
    n_chunks = -(-(blk_lo + (1 if n_hi else 0)) // cb)
    assert blk_lo >= cb

    mesh = plsc.VectorSubcoreMesh(core_axis_name="c", subcore_axis_name="s",
                                  num_cores=nc, num_subcores=ns)

    @functools.partial(
        pl.kernel,
        out_type=jax.ShapeDtypeStruct((2, 2, n_edges), jnp.float32),
        mesh=mesh,
        compiler_params=pltpu.CompilerParams(needs_layout_passes=False),
        scratch_types=[
            pltpu.VMEM((n_tab_words,), jnp.float32),
            pltpu.VMEM((ec,), jnp.int32),
            pltpu.VMEM((ec,), jnp.int32),
            pltpu.VMEM((ec,), jnp.float32),
            pltpu.VMEM((ec,), jnp.float32),
            pltpu.VMEM((ec,), jnp.float32),
            pltpu.VMEM((ec,), jnp.float32),
        ],
    )
    def sc_edge_kernel(tab_hbm, ei_hbm, out_hbm,
                       tab_v, src_v, dst_v, cb0, cb1, cb2, cb3):
        w = lax.axis_index("s") * nc + lax.axis_index("c")
        pltpu.sync_copy(tab_hbm, tab_v)
        blk0 = w * blk_lo + jnp.minimum(w, n_hi)
        my_blks = blk_lo + jnp.where(w < n_hi, 1, 0)

        for k in range(n_chunks):
            e0 = (blk0 + jnp.minimum(k * cb, my_blks - cb)) * 128
            pltpu.sync_copy(ei_hbm.at[0, pl.ds(e0, ec)], src_v)
            pltpu.sync_copy(ei_hbm.at[1, pl.ds(e0, ec)], dst_v)

            @plsc.parallel_loop(0, ec // 16, unroll=16)
            def group(g):
                s = src_v[pl.ds(g * 16, 16)]
                d = dst_v[pl.ds(g * 16, 16)]
                s8 = s * 8
                d8 = d * 8 + 4
                for c, buf in ((0, cb0), (1, cb1), (2, cb2), (3, cb3)):
                    a = plsc.load_gather(tab_v, [s8 + c])
                    b = plsc.load_gather(tab_v, [d8 + c])
                    v = a + b
                    t = jnp.exp(jnp.minimum(v + v, 60.0))
                    buf[pl.ds(g * 16, 16)] = (t - 1.0) / (t + 1.0)
            for c, buf in ((0, cb0), (1, cb1), (2, cb2), (3, cb3)):
                pltpu.sync_copy(buf, out_hbm.at[c // 2, c % 2, pl.ds(e0, ec)])

    return sc_edge_kernel


def kernel(x, edge_index, edge_types, node_types, W):
    n, d = x.shape
    e = edge_index.shape[1]
    wcat = jnp.concatenate([W[:d], W[d:2 * d]], axis=1)
    tcat = jnp.concatenate([W[2 * d:2 * d + 4],
                            W[2 * d + 4:2 * d + 8]], axis=1)

    table = pl.pallas_call(
        _table_body,
        out_shape=jax.ShapeDtypeStruct((n, 8), jnp.float32),
    )(x, node_types.reshape(n, 1), wcat, tcat)

    tab_flat = table.reshape(-1)
    out = _make_sc_edge_kernel(tab_flat.shape[0], e)(tab_flat, edge_index)
    return jnp.transpose(out, (2, 0, 1))


# speedup vs baseline: 1.7018x; 1.7018x over previous
import functools

import jax
import jax.numpy as jnp
from jax import lax
from jax.experimental import pallas as pl
from jax.experimental.pallas import tpu as pltpu
from jax.experimental.pallas import tpu_sc as plsc


def _table_body(x_ref, nt_ref, wcat_ref, tcat_ref, out_ref):
    acc = jnp.dot(x_ref[...], wcat_ref[...],
                  preferred_element_type=jnp.float32,
                  precision=lax.Precision.HIGHEST)
    n = acc.shape[0]
    nt = nt_ref[...].reshape(n, 1)
    for t in range(4):
        mask = jnp.where(nt == t, 1.0, 0.0)
        acc = acc + mask * tcat_ref[t:t + 1, :]
    out_ref[...] = acc


def _tanh_body(in_ref, out_ref):
    out_ref[...] = jnp.tanh(in_ref[...])


def _make_sc_edge_kernel(n_tab_words, n_edges):
    nc, ns = 2, 16
    nw = nc * ns
    assert n_edges % 128 == 0
    n_blk = n_edges // 128
    blk_lo = n_blk // nw
    n_hi = n_blk - blk_lo * nw
    cb = 16
    ec = cb * 128
    n_chunks = -(-(blk_lo + (1 if n_hi else 0)) // cb)
    assert blk_lo >= cb

    mesh = plsc.VectorSubcoreMesh(core_axis_name="c", subcore_axis_name="s",
                                  num_cores=nc, num_subcores=ns)

    @functools.partial(
        pl.kernel,
        out_type=jax.ShapeDtypeStruct((2, 2, n_edges), jnp.float32),
        mesh=mesh,
        compiler_params=pltpu.CompilerParams(needs_layout_passes=False),
        scratch_types=[
            pltpu.VMEM((n_tab_words,), jnp.float32),
            pltpu.VMEM((ec,), jnp.int32),
            pltpu.VMEM((ec,), jnp.int32),
            pltpu.VMEM((ec,), jnp.float32),
            pltpu.VMEM((ec,), jnp.float32),
            pltpu.VMEM((ec,), jnp.float32),
            pltpu.VMEM((ec,), jnp.float32),
        ],
    )
    def sc_edge_kernel(tab_hbm, ei_hbm, out_hbm,
                       tab_v, src_v, dst_v, cb0, cb1, cb2, cb3):
        w = lax.axis_index("s") * nc + lax.axis_index("c")
        pltpu.sync_copy(tab_hbm, tab_v)
        blk0 = w * blk_lo + jnp.minimum(w, n_hi)
        my_blks = blk_lo + jnp.where(w < n_hi, 1, 0)

        for k in range(n_chunks):
            e0 = (blk0 + jnp.minimum(k * cb, my_blks - cb)) * 128
            pltpu.sync_copy(ei_hbm.at[0, pl.ds(e0, ec)], src_v)
            pltpu.sync_copy(ei_hbm.at[1, pl.ds(e0, ec)], dst_v)

            @plsc.parallel_loop(0, ec // 16, unroll=8)
            def group(g):
                s = src_v[pl.ds(g * 16, 16)]
                d = dst_v[pl.ds(g * 16, 16)]
                s8 = s * 8
                d8 = d * 8 + 4
                for c, buf in ((0, cb0), (1, cb1), (2, cb2), (3, cb3)):
                    a = plsc.load_gather(tab_v, [s8 + c])
                    b = plsc.load_gather(tab_v, [d8 + c])
                    buf[pl.ds(g * 16, 16)] = a + b
            for c, buf in ((0, cb0), (1, cb1), (2, cb2), (3, cb3)):
                pltpu.sync_copy(buf, out_hbm.at[c // 2, c % 2, pl.ds(e0, ec)])

    return sc_edge_kernel


def kernel(x, edge_index, edge_types, node_types, W):
    n, d = x.shape
    e = edge_index.shape[1]
    wcat = jnp.concatenate([W[:d], W[d:2 * d]], axis=1)
    tcat = jnp.concatenate([W[2 * d:2 * d + 4],
                            W[2 * d + 4:2 * d + 8]], axis=1)

    table = pl.pallas_call(
        _table_body,
        out_shape=jax.ShapeDtypeStruct((n, 8), jnp.float32),
    )(x, node_types, wcat, tcat)

    tab_flat = table.reshape(-1)
    sums = _make_sc_edge_kernel(tab_flat.shape[0], e)(tab_flat, edge_index)

    grid = 10
    blk = e // grid
    out = pl.pallas_call(
        _tanh_body,
        grid=(grid,),
        in_specs=[pl.BlockSpec((2, 2, blk), lambda i: (0, 0, i))],
        out_specs=pl.BlockSpec((2, 2, blk), lambda i: (0, 0, i)),
        out_shape=jax.ShapeDtypeStruct((2, 2, e), jnp.float32),
    )(sums)
    return jnp.transpose(out, (2, 0, 1))


# speedup vs baseline: 1.9326x; 1.1356x over previous
import functools

import jax
import jax.numpy as jnp
from jax import lax
from jax.experimental import pallas as pl
from jax.experimental.pallas import tpu as pltpu
from jax.experimental.pallas import tpu_sc as plsc


def _table_body(x_ref, nt_ref, wcat_ref, tcat_ref, out_ref):
    acc = jnp.dot(x_ref[...], wcat_ref[...],
                  preferred_element_type=jnp.float32,
                  precision=lax.Precision.HIGHEST)
    n = acc.shape[0]
    nt = nt_ref[...].reshape(n, 1)
    for t in range(4):
        mask = jnp.where(nt == t, 1.0, 0.0)
        acc = acc + mask * tcat_ref[t:t + 1, :]
    out_ref[...] = acc


def _tanh_body(in_ref, out_ref):
    out_ref[...] = jnp.tanh(in_ref[...])


def _make_sc_edge_kernel(n_tab_words, n_edges):
    nc, ns = 2, 16
    nw = nc * ns
    assert n_edges % 128 == 0
    n_blk = n_edges // 128
    blk_lo = n_blk // nw
    n_hi = n_blk - blk_lo * nw
    cb = 16
    ec = cb * 128
    n_chunks = -(-(blk_lo + (1 if n_hi else 0)) // cb)
    assert blk_lo >= cb

    mesh = plsc.VectorSubcoreMesh(core_axis_name="c", subcore_axis_name="s",
                                  num_cores=nc, num_subcores=ns)

    @functools.partial(
        pl.kernel,
        out_type=jax.ShapeDtypeStruct((2, 2, n_edges), jnp.float32),
        mesh=mesh,
        compiler_params=pltpu.CompilerParams(needs_layout_passes=False),
        scratch_types=(
            [pltpu.VMEM((n_tab_words,), jnp.float32)]
            + [pltpu.VMEM((ec,), jnp.int32) for _ in range(4)]
            + [pltpu.VMEM((ec,), jnp.float32) for _ in range(8)]
            + [pltpu.SemaphoreType.DMA for _ in range(4)]
        ),
    )
    def sc_edge_kernel(tab_hbm, ei_hbm, out_hbm, tab_v,
                       s0, s1, d0, d1,
                       o00, o01, o02, o03, o10, o11, o12, o13,
                       semi0, semi1, semo0, semo1):
        w = lax.axis_index("s") * nc + lax.axis_index("c")
        blk0 = w * blk_lo + jnp.minimum(w, n_hi)
        my_blks = blk_lo + jnp.where(w < n_hi, 1, 0)
        semi = (semi0, semi1)
        semo = (semo0, semo1)
        srcs = (s0, s1)
        dsts = (d0, d1)
        obufs = ((o00, o01, o02, o03), (o10, o11, o12, o13))

        def chunk_base(k):
            return (blk0 + jnp.minimum(k * cb, my_blks - cb)) * 128

        def start_idx(k):
            b = k % 2
            e0 = chunk_base(k)
            return (
                pltpu.async_copy(ei_hbm.at[0, pl.ds(e0, ec)], srcs[b],
                                 semi[b]),
                pltpu.async_copy(ei_hbm.at[1, pl.ds(e0, ec)], dsts[b],
                                 semi[b]),
            )

        in_flight = start_idx(0)
        pltpu.sync_copy(tab_hbm, tab_v)
        out_flight = {0: (), 1: ()}

        for k in range(n_chunks):
            b = k % 2
            for cp in in_flight:
                cp.wait()
            if k + 1 < n_chunks:
                in_flight = start_idx(k + 1)
            for cp in out_flight[b]:
                cp.wait()
            out_flight[b] = ()

            sv = srcs[b]
            dv = dsts[b]
            bufs = obufs[b]

            @plsc.parallel_loop(0, ec // 16, unroll=8)
            def group(g):
                s = sv[pl.ds(g * 16, 16)]
                d = dv[pl.ds(g * 16, 16)]
                s8 = s * 8
                d8 = d * 8 + 4
                for c in range(4):
                    a = plsc.load_gather(tab_v, [s8 + c])
                    bb = plsc.load_gather(tab_v, [d8 + c])
                    bufs[c][pl.ds(g * 16, 16)] = a + bb

            e0 = chunk_base(k)
            out_flight[b] = tuple(
                pltpu.async_copy(bufs[c],
                                 out_hbm.at[c // 2, c % 2, pl.ds(e0, ec)],
                                 semo[b])
                for c in range(4))

        for b in (0, 1):
            for cp in out_flight[b]:
                cp.wait()

    return sc_edge_kernel


def kernel(x, edge_index, edge_types, node_types, W):
    n, d = x.shape
    e = edge_index.shape[1]
    wcat = jnp.concatenate([W[:d], W[d:2 * d]], axis=1)
    tcat = jnp.concatenate([W[2 * d:2 * d + 4],
                            W[2 * d + 4:2 * d + 8]], axis=1)

    table = pl.pallas_call(
        _table_body,
        out_shape=jax.ShapeDtypeStruct((n, 8), jnp.float32),
    )(x, node_types, wcat, tcat)

    tab_flat = table.reshape(-1)
    sums = _make_sc_edge_kernel(tab_flat.shape[0], e)(tab_flat, edge_index)

    grid = 10
    blk = e // grid
    out = pl.pallas_call(
        _tanh_body,
        grid=(grid,),
        in_specs=[pl.BlockSpec((2, 2, blk), lambda i: (0, 0, i))],
        out_specs=pl.BlockSpec((2, 2, blk), lambda i: (0, 0, i)),
        out_shape=jax.ShapeDtypeStruct((2, 2, e), jnp.float32),
    )(sums)
    return jnp.transpose(out, (2, 0, 1))


# speedup vs baseline: 1.9679x; 1.0183x over previous
import functools

import jax
import jax.numpy as jnp
from jax import lax
from jax.experimental import pallas as pl
from jax.experimental.pallas import tpu as pltpu
from jax.experimental.pallas import tpu_sc as plsc


def _table_body(x_ref, nt_ref, wcat_ref, tcat_ref, out_ref):
    acc = jnp.dot(x_ref[...], wcat_ref[...],
                  preferred_element_type=jnp.float32,
                  precision=lax.Precision.HIGHEST)
    n = acc.shape[0]
    nt = nt_ref[...].reshape(n, 1)
    b0 = (nt & 1) == 1
    b1 = nt >= 2
    lo = jnp.where(b0, tcat_ref[1:2, :], tcat_ref[0:1, :])
    hi = jnp.where(b0, tcat_ref[3:4, :], tcat_ref[2:3, :])
    out_ref[...] = acc + jnp.where(b1, hi, lo)


def _tanh_body(in_ref, out_ref):
    out_ref[...] = jnp.tanh(in_ref[...])


def _make_sc_edge_kernel(n_tab_words, n_edges):
    nc, ns = 2, 16
    nw = nc * ns
    assert n_edges % 128 == 0
    n_blk = n_edges // 128
    blk_lo = n_blk // nw
    n_hi = n_blk - blk_lo * nw
    cb = 16
    ec = cb * 128
    n_chunks = -(-(blk_lo + (1 if n_hi else 0)) // cb)
    assert blk_lo >= cb

    mesh = plsc.VectorSubcoreMesh(core_axis_name="c", subcore_axis_name="s",
                                  num_cores=nc, num_subcores=ns)

    @functools.partial(
        pl.kernel,
        out_type=jax.ShapeDtypeStruct((2, 2, n_edges), jnp.float32),
        mesh=mesh,
        compiler_params=pltpu.CompilerParams(needs_layout_passes=False),
        scratch_types=(
            [pltpu.VMEM((n_tab_words,), jnp.float32)]
            + [pltpu.VMEM((ec,), jnp.int32) for _ in range(4)]
            + [pltpu.VMEM((ec,), jnp.float32) for _ in range(8)]
            + [pltpu.SemaphoreType.DMA for _ in range(4)]
        ),
    )
    def sc_edge_kernel(tab_hbm, ei_hbm, out_hbm, tab_v,
                       s0, s1, d0, d1,
                       o00, o01, o02, o03, o10, o11, o12, o13,
                       semi0, semi1, semo0, semo1):
        w = lax.axis_index("s") * nc + lax.axis_index("c")
        blk0 = w * blk_lo + jnp.minimum(w, n_hi)
        my_blks = blk_lo + jnp.where(w < n_hi, 1, 0)
        semi = (semi0, semi1)
        semo = (semo0, semo1)
        srcs = (s0, s1)
        dsts = (d0, d1)
        obufs = ((o00, o01, o02, o03), (o10, o11, o12, o13))

        def chunk_base(k):
            return (blk0 + jnp.minimum(k * cb, my_blks - cb)) * 128

        def start_idx(k):
            b = k % 2
            e0 = chunk_base(k)
            return (
                pltpu.async_copy(ei_hbm.at[0, pl.ds(e0, ec)], srcs[b],
                                 semi[b]),
                pltpu.async_copy(ei_hbm.at[1, pl.ds(e0, ec)], dsts[b],
                                 semi[b]),
            )

        in_flight = start_idx(0)
        pltpu.sync_copy(tab_hbm, tab_v)
        out_flight = {0: (), 1: ()}

        for k in range(n_chunks):
            b = k % 2
            for cp in in_flight:
                cp.wait()
            if k + 1 < n_chunks:
                in_flight = start_idx(k + 1)
            for cp in out_flight[b]:
                cp.wait()
            out_flight[b] = ()

            sv = srcs[b]
            dv = dsts[b]
            bufs = obufs[b]

            @plsc.parallel_loop(0, ec // 16, unroll=8)
            def group(g):
                s = sv[pl.ds(g * 16, 16)]
                d = dv[pl.ds(g * 16, 16)]
                s8 = s * 8
                d8 = d * 8 + 4
                for c in range(4):
                    a = plsc.load_gather(tab_v, [s8 + c])
                    bb = plsc.load_gather(tab_v, [d8 + c])
                    bufs[c][pl.ds(g * 16, 16)] = a + bb

            e0 = chunk_base(k)
            out_flight[b] = tuple(
                pltpu.async_copy(bufs[c],
                                 out_hbm.at[c // 2, c % 2, pl.ds(e0, ec)],
                                 semo[b])
                for c in range(4))

        for b in (0, 1):
            for cp in out_flight[b]:
                cp.wait()

    return sc_edge_kernel


def kernel(x, edge_index, edge_types, node_types, W):
    n, d = x.shape
    e = edge_index.shape[1]
    wcat = jnp.concatenate([W[:d], W[d:2 * d]], axis=1)
    tcat = jnp.concatenate([W[2 * d:2 * d + 4],
                            W[2 * d + 4:2 * d + 8]], axis=1)

    table = pl.pallas_call(
        _table_body,
        out_shape=jax.ShapeDtypeStruct((n, 8), jnp.float32),
    )(x, node_types, wcat, tcat)

    tab_flat = table.reshape(-1)
    sums = _make_sc_edge_kernel(tab_flat.shape[0], e)(tab_flat, edge_index)

    grid = 10
    blk = e // grid
    out = pl.pallas_call(
        _tanh_body,
        grid=(grid,),
        in_specs=[pl.BlockSpec((2, 2, blk), lambda i: (0, 0, i))],
        out_specs=pl.BlockSpec((2, 2, blk), lambda i: (0, 0, i)),
        out_shape=jax.ShapeDtypeStruct((2, 2, e), jnp.float32),
    )(sums)
    return jnp.transpose(out, (2, 0, 1))


# speedup vs baseline: 2.3312x; 1.1846x over previous
import functools

import jax
import jax.numpy as jnp
from jax import lax
from jax.experimental import pallas as pl
from jax.experimental.pallas import tpu as pltpu
from jax.experimental.pallas import tpu_sc as plsc


def _table_body(x_ref, nt_ref, wcatt_ref, tcatt_ref, out_ref):
    acc = lax.dot_general(wcatt_ref[...], x_ref[...],
                          (((1,), (1,)), ((), ())),
                          preferred_element_type=jnp.float32,
                          precision=lax.Precision.HIGHEST)
    nt = nt_ref[...].reshape(1, -1)
    b0 = (nt & 1) == 1
    b1 = nt >= 2
    lo = jnp.where(b0, tcatt_ref[:, 1:2], tcatt_ref[:, 0:1])
    hi = jnp.where(b0, tcatt_ref[:, 3:4], tcatt_ref[:, 2:3])
    out_ref[...] = acc + jnp.where(b1, hi, lo)


def _tanh_body(in_ref, out_ref):
    out_ref[...] = jnp.tanh(in_ref[...])


def _make_sc_edge_kernel(n_tab_words, n_edges, n_nodes):
    nc, ns = 2, 16
    nw = nc * ns
    assert n_edges % 128 == 0
    n_blk = n_edges // 128
    blk_lo = n_blk // nw
    n_hi = n_blk - blk_lo * nw
    cb = 16
    ec = cb * 128
    n_chunks = -(-(blk_lo + (1 if n_hi else 0)) // cb)
    assert blk_lo >= cb

    mesh = plsc.VectorSubcoreMesh(core_axis_name="c", subcore_axis_name="s",
                                  num_cores=nc, num_subcores=ns)

    @functools.partial(
        pl.kernel,
        out_type=jax.ShapeDtypeStruct((2, 2, n_edges), jnp.float32),
        mesh=mesh,
        compiler_params=pltpu.CompilerParams(needs_layout_passes=False),
        scratch_types=(
            [pltpu.VMEM((n_tab_words,), jnp.float32)]
            + [pltpu.VMEM((ec,), jnp.int32) for _ in range(4)]
            + [pltpu.VMEM((ec,), jnp.float32) for _ in range(8)]
            + [pltpu.SemaphoreType.DMA for _ in range(4)]
        ),
    )
    def sc_edge_kernel(tab_hbm, ei_hbm, out_hbm, tab_v,
                       s0, s1, d0, d1,
                       o00, o01, o02, o03, o10, o11, o12, o13,
                       semi0, semi1, semo0, semo1):
        w = lax.axis_index("s") * nc + lax.axis_index("c")
        blk0 = w * blk_lo + jnp.minimum(w, n_hi)
        my_blks = blk_lo + jnp.where(w < n_hi, 1, 0)
        semi = (semi0, semi1)
        semo = (semo0, semo1)
        srcs = (s0, s1)
        dsts = (d0, d1)
        obufs = ((o00, o01, o02, o03), (o10, o11, o12, o13))

        def chunk_base(k):
            return (blk0 + jnp.minimum(k * cb, my_blks - cb)) * 128

        def start_idx(k):
            b = k % 2
            e0 = chunk_base(k)
            return (
                pltpu.async_copy(ei_hbm.at[0, pl.ds(e0, ec)], srcs[b],
                                 semi[b]),
                pltpu.async_copy(ei_hbm.at[1, pl.ds(e0, ec)], dsts[b],
                                 semi[b]),
            )

        in_flight = start_idx(0)
        pltpu.sync_copy(tab_hbm, tab_v)
        out_flight = {0: (), 1: ()}

        for k in range(n_chunks):
            b = k % 2
            for cp in in_flight:
                cp.wait()
            if k + 1 < n_chunks:
                in_flight = start_idx(k + 1)
            for cp in out_flight[b]:
                cp.wait()
            out_flight[b] = ()

            sv = srcs[b]
            dv = dsts[b]
            bufs = obufs[b]

            @plsc.parallel_loop(0, ec // 16, unroll=8)
            def group(g):
                s = sv[pl.ds(g * 16, 16)]
                d = dv[pl.ds(g * 16, 16)]
                for c in range(4):
                    a = plsc.load_gather(tab_v, [s + c * n_nodes])
                    bb = plsc.load_gather(tab_v, [d + (4 + c) * n_nodes])
                    bufs[c][pl.ds(g * 16, 16)] = a + bb

            e0 = chunk_base(k)
            out_flight[b] = tuple(
                pltpu.async_copy(bufs[c],
                                 out_hbm.at[c // 2, c % 2, pl.ds(e0, ec)],
                                 semo[b])
                for c in range(4))

        for b in (0, 1):
            for cp in out_flight[b]:
                cp.wait()

    return sc_edge_kernel


def kernel(x, edge_index, edge_types, node_types, W):
    n, d = x.shape
    e = edge_index.shape[1]
    wcatt = jnp.concatenate([W[:d], W[d:2 * d]], axis=1).T
    tcatt = jnp.concatenate([W[2 * d:2 * d + 4],
                             W[2 * d + 4:2 * d + 8]], axis=1).T

    table = pl.pallas_call(
        _table_body,
        out_shape=jax.ShapeDtypeStruct((8, n), jnp.float32),
    )(x, node_types, wcatt, tcatt)

    tab_flat = table.reshape(-1)
    sums = _make_sc_edge_kernel(tab_flat.shape[0], e, n)(tab_flat, edge_index)

    grid = 10
    blk = e // grid
    out = pl.pallas_call(
        _tanh_body,
        grid=(grid,),
        in_specs=[pl.BlockSpec((2, 2, blk), lambda i: (0, 0, i))],
        out_specs=pl.BlockSpec((2, 2, blk), lambda i: (0, 0, i)),
        out_shape=jax.ShapeDtypeStruct((2, 2, e), jnp.float32),
    )(sums)
    return jnp.transpose(out, (2, 0, 1))
